# Initial kernel scaffold; baseline (speedup 1.0000x reference)
import functools

import jax
import jax.numpy as jnp
from jax import lax
from jax.experimental import pallas as pl
from jax.experimental.pallas import tpu as pltpu
from jax.experimental.pallas import tpu_sc as plsc

_B, _S, _D = 1, 2048, 768
_E, _K, _F, _R = 8, 2, 1536, 16
_T = _B * _S
_BLK = 128
_NB = (_K * _T + _E * _BLK) // _BLK
_P = _NB * _BLK
_LANES = 128

_NC, _NS = 2, 16
_NW = _NC * _NS


def _router_body(x_ref, wr_ref, ids_ref, ws_ref):
    xb = x_ref[...]
    logits = jnp.dot(xb, wr_ref[...], preferred_element_type=jnp.float32)
    rows = xb.shape[0]
    lane = lax.broadcasted_iota(jnp.int32, (rows, _LANES), 1)
    valid = lane < _E
    l = jnp.where(valid, logits, -1e30)
    m = jnp.max(l, axis=1, keepdims=True)
    p = jnp.exp(l - m)
    p = jnp.where(valid, p, 0.0)
    p = p / jnp.sum(p, axis=1, keepdims=True)
    m1 = jnp.max(p, axis=1, keepdims=True)
    i1 = jnp.min(jnp.where((p == m1) & valid, lane, _LANES), axis=1,
                 keepdims=True)
    p2 = jnp.where(lane == i1, -1.0, p)
    m2 = jnp.max(p2, axis=1, keepdims=True)
    i2 = jnp.min(jnp.where((p2 == m2) & valid, lane, _LANES), axis=1,
                 keepdims=True)
    denom = m1 + m2
    ids_ref[...] = jnp.where(lane == 0, i1, jnp.where(lane == 1, i2, 0))
    ws_ref[...] = jnp.where(lane == 0, m1 / denom,
                            jnp.where(lane == 1, m2 / denom, 0.0))


def _run_router(x, w_router_pad, interpret=False):
    rows = 512
    return pl.pallas_call(
        _router_body,
        grid=(_T // rows,),
        in_specs=[
            pl.BlockSpec((rows, _D), lambda i: (i, 0)),
            pl.BlockSpec((_D, _LANES), lambda i: (0, 0)),
        ],
        out_specs=[
            pl.BlockSpec((rows, _LANES), lambda i: (i, 0)),
            pl.BlockSpec((rows, _LANES), lambda i: (i, 0)),
        ],
        out_shape=[
            jax.ShapeDtypeStruct((_T, _LANES), jnp.int32),
            jax.ShapeDtypeStruct((_T, _LANES), jnp.float32),
        ],
        interpret=interpret,
    )(x, w_router_pad)


def _routing_metadata(ids, ws):
    fid = ids.reshape(-1)
    fw = ws.reshape(-1)
    sort_idx = jnp.argsort(fid, stable=True)
    sorted_e = fid[sort_idx]
    counts = jnp.zeros((_E,), jnp.int32).at[fid].add(1)
    nb_e = (counts + _BLK - 1) // _BLK
    cum_nb = jnp.cumsum(nb_e)
    pstart = jnp.concatenate([jnp.zeros((1,), jnp.int32),
                              cum_nb[:-1] * _BLK])
    starts = jnp.concatenate([jnp.zeros((1,), jnp.int32),
                              jnp.cumsum(counts)[:-1]])
    rank = jnp.arange(_T * _K, dtype=jnp.int32) - starts[sorted_e]
    ppos = pstart[sorted_e] + rank
    src_tok = (sort_idx // _K).astype(jnp.int32)
    perm = jnp.zeros((_P,), jnp.int32).at[ppos].set(src_tok)
    wperm = jnp.zeros((_P,), jnp.float32).at[ppos].set(fw[sort_idx])
    posflat = jnp.zeros((_T * _K,), jnp.int32).at[sort_idx].set(ppos)
    pos = posflat.reshape(_T, _K)
    blk = jnp.arange(_NB, dtype=jnp.int32)
    block_expert = jnp.clip(
        jnp.sum((blk[:, None] >= cum_nb[None, :]).astype(jnp.int32), axis=1),
        0, _E - 1).astype(jnp.int32)
    return perm, wperm, pos[:, 0], pos[:, 1], block_expert


_SC_MESH = plsc.VectorSubcoreMesh(core_axis_name="c", subcore_axis_name="s")


def _sc_gather_body(x_hbm, idx_hbm, out_hbm, idx_v, rows_v, sem):
    wid = lax.axis_index("s") * _NC + lax.axis_index("c")
    rpw = _P // _NW
    base = wid * rpw
    pltpu.sync_copy(idx_hbm.at[pl.ds(base, rpw)], idx_v)
    pltpu.async_copy(x_hbm.at[idx_v], rows_v, sem).wait()
    pltpu.sync_copy(rows_v, out_hbm.at[pl.ds(base, rpw)])


_sc_gather = functools.partial(
    pl.kernel,
    out_type=jax.ShapeDtypeStruct((_P, _D), jnp.float32),
    mesh=_SC_MESH,
    scratch_types=[
        pltpu.VMEM((_P // _NW,), jnp.int32),
        pltpu.VMEM((_P // _NW, _D), jnp.float32),
        pltpu.SemaphoreType.DMA,
    ],
)(_sc_gather_body)


def _ffn_body(be_ref, xs_ref, wp_ref, wg_ref, wu_ref, wd_ref,
              ga_ref, gb_ref, ua_ref, ub_ref, da_ref, db_ref, out_ref):
    del be_ref
    xb = xs_ref[...]
    c11 = (((1,), (1,)), ((), ()))
    g = jnp.dot(xb, wg_ref[0], preferred_element_type=jnp.float32)
    xag = lax.dot_general(xb, ga_ref[0], c11,
                          preferred_element_type=jnp.float32)
    g = g + lax.dot_general(xag, gb_ref[0], c11,
                            preferred_element_type=jnp.float32)
    u = jnp.dot(xb, wu_ref[0], preferred_element_type=jnp.float32)
    xau = lax.dot_general(xb, ua_ref[0], c11,
                          preferred_element_type=jnp.float32)
    u = u + lax.dot_general(xau, ub_ref[0], c11,
                            preferred_element_type=jnp.float32)
    h = (g * jax.nn.sigmoid(g)) * u
    y = jnp.dot(h, wd_ref[0], preferred_element_type=jnp.float32)
    had = lax.dot_general(h, da_ref[0], c11,
                          preferred_element_type=jnp.float32)
    y = y + lax.dot_general(had, db_ref[0], c11,
                            preferred_element_type=jnp.float32)
    out_ref[...] = y * wp_ref[...]


def _run_ffn(block_expert, xs, wperm2d, w_gate, w_up, w_down,
             ga, gb, ua, ub, da, db, interpret=False):
    def eix(b, be):
        return (be[b], 0, 0)

    grid_spec = pltpu.PrefetchScalarGridSpec(
        num_scalar_prefetch=1,
        grid=(_NB,),
        in_specs=[
            pl.BlockSpec((_BLK, _D), lambda b, be: (b, 0)),
            pl.BlockSpec((_BLK, 1), lambda b, be: (b, 0)),
            pl.BlockSpec((1, _D, _F), eix),
            pl.BlockSpec((1, _D, _F), eix),
            pl.BlockSpec((1, _F, _D), eix),
            pl.BlockSpec((1, _R, _D), eix),
            pl.BlockSpec((1, _F, _R), eix),
            pl.BlockSpec((1, _R, _D), eix),
            pl.BlockSpec((1, _F, _R), eix),
            pl.BlockSpec((1, _R, _F), eix),
            pl.BlockSpec((1, _D, _R), eix),
        ],
        out_specs=pl.BlockSpec((_BLK, _D), lambda b, be: (b, 0)),
    )
    return pl.pallas_call(
        _ffn_body,
        grid_spec=grid_spec,
        out_shape=jax.ShapeDtypeStruct((_P, _D), jnp.float32),
        compiler_params=pltpu.CompilerParams(
            dimension_semantics=("arbitrary",),
        ),
        interpret=interpret,
    )(block_expert, xs, wperm2d, w_gate, w_up, w_down,
      ga, gb, ua, ub, da, db)


def _sc_combine_body(ys_hbm, pos0_hbm, pos1_hbm, out_hbm,
                     p0_v, p1_v, acc_v, sem):
    wid = lax.axis_index("s") * _NC + lax.axis_index("c")
    tpw = _T // _NW
    base = wid * tpw
    pltpu.sync_copy(pos0_hbm.at[pl.ds(base, tpw)], p0_v)
    pltpu.sync_copy(pos1_hbm.at[pl.ds(base, tpw)], p1_v)
    pltpu.async_copy(ys_hbm.at[p0_v], acc_v, sem).wait()
    pltpu.async_copy(ys_hbm.at[p1_v], acc_v, sem, add=True).wait()
    pltpu.sync_copy(acc_v, out_hbm.at[pl.ds(base, tpw)])


_sc_combine = functools.partial(
    pl.kernel,
    out_type=jax.ShapeDtypeStruct((_T, _D), jnp.float32),
    mesh=_SC_MESH,
    scratch_types=[
        pltpu.VMEM((_T // _NW,), jnp.int32),
        pltpu.VMEM((_T // _NW,), jnp.int32),
        pltpu.VMEM((_T // _NW, _D), jnp.float32),
        pltpu.SemaphoreType.DMA,
    ],
)(_sc_combine_body)


def kernel(hidden_states, W_router, W_gate, W_up, W_down,
           gate_lora_a, gate_lora_b, up_lora_a, up_lora_b,
           down_lora_a, down_lora_b):
    x = hidden_states.reshape(_T, _D)
    wr_pad = jnp.zeros((_D, _LANES), jnp.float32).at[:, :_E].set(W_router)
    ids_full, ws_full = _run_router(x, wr_pad)
    ids = ids_full[:, :_K]
    ws = ws_full[:, :_K]
    perm, wperm, pos0, pos1, block_expert = _routing_metadata(ids, ws)
    xs = _sc_gather(x, perm)
    ys = _run_ffn(block_expert, xs, wperm.reshape(_P, 1),
                  W_gate, W_up, W_down,
                  gate_lora_a, gate_lora_b, up_lora_a, up_lora_b,
                  down_lora_a, down_lora_b)
    out = _sc_combine(ys, pos0, pos1)
    return out.reshape(_B, _S, _D)


# speedup vs baseline: 1.1725x; 1.1725x over previous
import functools

import jax
import jax.numpy as jnp
from jax import lax
from jax.experimental import pallas as pl
from jax.experimental.pallas import tpu as pltpu
from jax.experimental.pallas import tpu_sc as plsc

_B, _S, _D = 1, 2048, 768
_E, _K, _F, _R = 8, 2, 1536, 16
_T = _B * _S
_BLK = 128
_NB = (_K * _T + _E * _BLK) // _BLK
_P = _NB * _BLK
_LANES = 128

_NC, _NS = 2, 16
_NW = _NC * _NS


def _router_body(x_ref, wr_ref, ids_ref, ws_ref):
    xb = x_ref[...]
    logits = jnp.dot(xb, wr_ref[...], preferred_element_type=jnp.float32)
    rows = xb.shape[0]
    lane = lax.broadcasted_iota(jnp.int32, (rows, _LANES), 1)
    valid = lane < _E
    l = jnp.where(valid, logits, -1e30)
    m = jnp.max(l, axis=1, keepdims=True)
    p = jnp.exp(l - m)
    p = jnp.where(valid, p, 0.0)
    p = p / jnp.sum(p, axis=1, keepdims=True)
    m1 = jnp.max(p, axis=1, keepdims=True)
    i1 = jnp.min(jnp.where((p == m1) & valid, lane, _LANES), axis=1,
                 keepdims=True)
    p2 = jnp.where(lane == i1, -1.0, p)
    m2 = jnp.max(p2, axis=1, keepdims=True)
    i2 = jnp.min(jnp.where((p2 == m2) & valid, lane, _LANES), axis=1,
                 keepdims=True)
    denom = m1 + m2
    ids_ref[...] = jnp.where(lane == 0, i1, jnp.where(lane == 1, i2, 0))
    ws_ref[...] = jnp.where(lane == 0, m1 / denom,
                            jnp.where(lane == 1, m2 / denom, 0.0))


def _run_router(x, w_router_pad, interpret=False):
    rows = 512
    return pl.pallas_call(
        _router_body,
        grid=(_T // rows,),
        in_specs=[
            pl.BlockSpec((rows, _D), lambda i: (i, 0)),
            pl.BlockSpec((_D, _LANES), lambda i: (0, 0)),
        ],
        out_specs=[
            pl.BlockSpec((rows, _LANES), lambda i: (i, 0)),
            pl.BlockSpec((rows, _LANES), lambda i: (i, 0)),
        ],
        out_shape=[
            jax.ShapeDtypeStruct((_T, _LANES), jnp.int32),
            jax.ShapeDtypeStruct((_T, _LANES), jnp.float32),
        ],
        interpret=interpret,
    )(x, w_router_pad)


def _routing_metadata(ids, ws):
    fid = ids.reshape(-1)
    fw = ws.reshape(-1)
    sort_idx = jnp.argsort(fid, stable=True)
    sorted_e = fid[sort_idx]
    counts = jnp.zeros((_E,), jnp.int32).at[fid].add(1)
    nb_e = (counts + _BLK - 1) // _BLK
    cum_nb = jnp.cumsum(nb_e)
    pstart = jnp.concatenate([jnp.zeros((1,), jnp.int32),
                              cum_nb[:-1] * _BLK])
    starts = jnp.concatenate([jnp.zeros((1,), jnp.int32),
                              jnp.cumsum(counts)[:-1]])
    rank = jnp.arange(_T * _K, dtype=jnp.int32) - starts[sorted_e]
    ppos = pstart[sorted_e] + rank
    src_tok = (sort_idx // _K).astype(jnp.int32)
    perm = jnp.zeros((_P,), jnp.int32).at[ppos].set(src_tok)
    wperm = jnp.zeros((_P,), jnp.float32).at[ppos].set(fw[sort_idx])
    posflat = jnp.zeros((_T * _K,), jnp.int32).at[sort_idx].set(ppos)
    pos = posflat.reshape(_T, _K)
    blk = jnp.arange(_NB, dtype=jnp.int32)
    block_expert = jnp.clip(
        jnp.sum((blk[:, None] >= cum_nb[None, :]).astype(jnp.int32), axis=1),
        0, _E - 1).astype(jnp.int32)
    return perm, wperm, pos[:, 0], pos[:, 1], block_expert


def _sc_gather_body(x_hbm, idx_hbm, out_hbm, idx_v, rows_v, sem):
    wid = lax.axis_index("s") * _NC + lax.axis_index("c")
    rpw = _P // _NW
    base = wid * rpw
    pltpu.sync_copy(idx_hbm.at[pl.ds(base, rpw)], idx_v)
    pltpu.async_copy(x_hbm.at[idx_v], rows_v, sem).wait()
    pltpu.sync_copy(rows_v, out_hbm.at[pl.ds(base, rpw)])


@functools.cache
def _sc_gather():
    return pl.kernel(
        _sc_gather_body,
        out_type=jax.ShapeDtypeStruct((_P, _D), jnp.float32),
        mesh=plsc.VectorSubcoreMesh(core_axis_name="c", subcore_axis_name="s"),
        scratch_types=[
            pltpu.VMEM((_P // _NW,), jnp.int32),
            pltpu.VMEM((_P // _NW, _D), jnp.float32),
            pltpu.SemaphoreType.DMA,
        ],
    )


def _ffn_body(be_ref, xs_ref, wp_ref, wg_ref, wu_ref, wd_ref,
              ga_ref, gb_ref, ua_ref, ub_ref, da_ref, db_ref, out_ref):
    del be_ref
    xb = xs_ref[...]
    c11 = (((1,), (1,)), ((), ()))
    g = jnp.dot(xb, wg_ref[0], preferred_element_type=jnp.float32)
    xag = lax.dot_general(xb, ga_ref[0], c11,
                          preferred_element_type=jnp.float32)
    g = g + lax.dot_general(xag, gb_ref[0], c11,
                            preferred_element_type=jnp.float32)
    u = jnp.dot(xb, wu_ref[0], preferred_element_type=jnp.float32)
    xau = lax.dot_general(xb, ua_ref[0], c11,
                          preferred_element_type=jnp.float32)
    u = u + lax.dot_general(xau, ub_ref[0], c11,
                            preferred_element_type=jnp.float32)
    h = (g * jax.nn.sigmoid(g)) * u
    y = jnp.dot(h, wd_ref[0], preferred_element_type=jnp.float32)
    had = lax.dot_general(h, da_ref[0], c11,
                          preferred_element_type=jnp.float32)
    y = y + lax.dot_general(had, db_ref[0], c11,
                            preferred_element_type=jnp.float32)
    out_ref[...] = y * wp_ref[...]


def _run_ffn(block_expert, xs, wperm2d, w_gate, w_up, w_down,
             ga, gb, ua, ub, da, db, interpret=False):
    def eix(b, be):
        return (be[b], 0, 0)

    grid_spec = pltpu.PrefetchScalarGridSpec(
        num_scalar_prefetch=1,
        grid=(_NB,),
        in_specs=[
            pl.BlockSpec((_BLK, _D), lambda b, be: (b, 0)),
            pl.BlockSpec((_BLK, 1), lambda b, be: (b, 0)),
            pl.BlockSpec((1, _D, _F), eix),
            pl.BlockSpec((1, _D, _F), eix),
            pl.BlockSpec((1, _F, _D), eix),
            pl.BlockSpec((1, _R, _D), eix),
            pl.BlockSpec((1, _F, _R), eix),
            pl.BlockSpec((1, _R, _D), eix),
            pl.BlockSpec((1, _F, _R), eix),
            pl.BlockSpec((1, _R, _F), eix),
            pl.BlockSpec((1, _D, _R), eix),
        ],
        out_specs=pl.BlockSpec((_BLK, _D), lambda b, be: (b, 0)),
    )
    return pl.pallas_call(
        _ffn_body,
        grid_spec=grid_spec,
        out_shape=jax.ShapeDtypeStruct((_P, _D), jnp.float32),
        compiler_params=pltpu.CompilerParams(
            dimension_semantics=("arbitrary",),
        ),
        interpret=interpret,
    )(block_expert, xs, wperm2d, w_gate, w_up, w_down,
      ga, gb, ua, ub, da, db)


def _sc_combine_body(ys_hbm, pos0_hbm, pos1_hbm, out_hbm,
                     p0_v, p1_v, acc_v, b1_v, sem0, sem1):
    wid = lax.axis_index("s") * _NC + lax.axis_index("c")
    tpw = _T // _NW
    base = wid * tpw
    pltpu.sync_copy(pos0_hbm.at[pl.ds(base, tpw)], p0_v)
    pltpu.sync_copy(pos1_hbm.at[pl.ds(base, tpw)], p1_v)
    cp0 = pltpu.async_copy(ys_hbm.at[p0_v], acc_v, sem0)
    cp1 = pltpu.async_copy(ys_hbm.at[p1_v], b1_v, sem1)
    cp0.wait()
    cp1.wait()

    nv = _D // 16

    def tok(t, carry):
        for j in range(nv):
            sl = pl.ds(j * 16, 16)
            acc_v[t, sl] += b1_v[t, sl]
        return carry

    lax.fori_loop(0, tpw, tok, 0, unroll=False)
    pltpu.sync_copy(acc_v, out_hbm.at[pl.ds(base, tpw)])


@functools.cache
def _sc_combine():
    return pl.kernel(
        _sc_combine_body,
        out_type=jax.ShapeDtypeStruct((_T, _D), jnp.float32),
        mesh=plsc.VectorSubcoreMesh(core_axis_name="c", subcore_axis_name="s"),
        scratch_types=[
            pltpu.VMEM((_T // _NW,), jnp.int32),
            pltpu.VMEM((_T // _NW,), jnp.int32),
            pltpu.VMEM((_T // _NW, _D), jnp.float32),
            pltpu.VMEM((_T // _NW, _D), jnp.float32),
            pltpu.SemaphoreType.DMA,
            pltpu.SemaphoreType.DMA,
        ],
    )


def kernel(hidden_states, W_router, W_gate, W_up, W_down,
           gate_lora_a, gate_lora_b, up_lora_a, up_lora_b,
           down_lora_a, down_lora_b):
    x = hidden_states.reshape(_T, _D)
    wr_pad = jnp.zeros((_D, _LANES), jnp.float32).at[:, :_E].set(W_router)
    ids_full, ws_full = _run_router(x, wr_pad)
    ids = ids_full[:, :_K]
    ws = ws_full[:, :_K]
    perm, wperm, pos0, pos1, block_expert = _routing_metadata(ids, ws)
    xs = _sc_gather()(x, perm)
    ys = _run_ffn(block_expert, xs, wperm.reshape(_P, 1),
                  W_gate, W_up, W_down,
                  gate_lora_a, gate_lora_b, up_lora_a, up_lora_b,
                  down_lora_a, down_lora_b)
    out = _sc_combine()(ys, pos0, pos1)
    return out.reshape(_B, _S, _D)


# speedup vs baseline: 1.1733x; 1.0006x over previous
import functools

import jax
import jax.numpy as jnp
from jax import lax
from jax.experimental import pallas as pl
from jax.experimental.pallas import tpu as pltpu
from jax.experimental.pallas import tpu_sc as plsc

_B, _S, _D = 1, 2048, 768
_E, _K, _F, _R = 8, 2, 1536, 16
_T = _B * _S
_BLK = 128
_NB = (_K * _T + _E * _BLK) // _BLK
_P = _NB * _BLK
_LANES = 128

_NC, _NS = 2, 16
_NW = _NC * _NS


def _router_body(x_ref, wr_ref, ids_ref, ws_ref):
    xb = x_ref[...]
    logits = jnp.dot(xb, wr_ref[...], preferred_element_type=jnp.float32)
    rows = xb.shape[0]
    lane = lax.broadcasted_iota(jnp.int32, (rows, _LANES), 1)
    valid = lane < _E
    l = jnp.where(valid, logits, -1e30)
    m = jnp.max(l, axis=1, keepdims=True)
    p = jnp.exp(l - m)
    p = jnp.where(valid, p, 0.0)
    p = p / jnp.sum(p, axis=1, keepdims=True)
    m1 = jnp.max(p, axis=1, keepdims=True)
    i1 = jnp.min(jnp.where((p == m1) & valid, lane, _LANES), axis=1,
                 keepdims=True)
    p2 = jnp.where(lane == i1, -1.0, p)
    m2 = jnp.max(p2, axis=1, keepdims=True)
    i2 = jnp.min(jnp.where((p2 == m2) & valid, lane, _LANES), axis=1,
                 keepdims=True)
    denom = m1 + m2
    ids_ref[...] = jnp.where(lane == 0, i1, jnp.where(lane == 1, i2, 0))
    ws_ref[...] = jnp.where(lane == 0, m1 / denom,
                            jnp.where(lane == 1, m2 / denom, 0.0))


def _run_router(x, w_router_pad, interpret=False):
    rows = 512
    return pl.pallas_call(
        _router_body,
        grid=(_T // rows,),
        in_specs=[
            pl.BlockSpec((rows, _D), lambda i: (i, 0)),
            pl.BlockSpec((_D, _LANES), lambda i: (0, 0)),
        ],
        out_specs=[
            pl.BlockSpec((rows, _LANES), lambda i: (i, 0)),
            pl.BlockSpec((rows, _LANES), lambda i: (i, 0)),
        ],
        out_shape=[
            jax.ShapeDtypeStruct((_T, _LANES), jnp.int32),
            jax.ShapeDtypeStruct((_T, _LANES), jnp.float32),
        ],
        interpret=interpret,
    )(x, w_router_pad)


def _routing_metadata(ids, ws):
    fid = ids.reshape(-1)
    fw = ws.reshape(-1)
    sort_idx = jnp.argsort(fid, stable=True)
    sorted_e = fid[sort_idx]
    counts = jnp.zeros((_E,), jnp.int32).at[fid].add(1)
    nb_e = (counts + _BLK - 1) // _BLK
    cum_nb = jnp.cumsum(nb_e)
    pstart = jnp.concatenate([jnp.zeros((1,), jnp.int32),
                              cum_nb[:-1] * _BLK])
    starts = jnp.concatenate([jnp.zeros((1,), jnp.int32),
                              jnp.cumsum(counts)[:-1]])
    rank = jnp.arange(_T * _K, dtype=jnp.int32) - starts[sorted_e]
    ppos = pstart[sorted_e] + rank
    src_tok = (sort_idx // _K).astype(jnp.int32)
    perm = jnp.zeros((_P,), jnp.int32).at[ppos].set(src_tok)
    wperm = jnp.zeros((_P,), jnp.float32).at[ppos].set(fw[sort_idx])
    posflat = jnp.zeros((_T * _K,), jnp.int32).at[sort_idx].set(ppos)
    pos = posflat.reshape(_T, _K)
    blk = jnp.arange(_NB, dtype=jnp.int32)
    block_expert = jnp.clip(
        jnp.sum((blk[:, None] >= cum_nb[None, :]).astype(jnp.int32), axis=1),
        0, _E - 1).astype(jnp.int32)
    return perm, wperm, pos[:, 0], pos[:, 1], block_expert


def _sc_gather_body(x_hbm, idx_hbm, out_hbm, idx_v, rows_v, sem):
    wid = lax.axis_index("s") * _NC + lax.axis_index("c")
    rpw = _P // _NW
    base = wid * rpw
    pltpu.sync_copy(idx_hbm.at[pl.ds(base, rpw)], idx_v)
    pltpu.async_copy(x_hbm.at[idx_v], rows_v, sem).wait()
    pltpu.sync_copy(rows_v, out_hbm.at[pl.ds(base, rpw)])


@functools.cache
def _sc_gather():
    return pl.kernel(
        _sc_gather_body,
        out_type=jax.ShapeDtypeStruct((_P, _D), jnp.float32),
        mesh=plsc.VectorSubcoreMesh(core_axis_name="c", subcore_axis_name="s"),
        scratch_types=[
            pltpu.VMEM((_P // _NW,), jnp.int32),
            pltpu.VMEM((_P // _NW, _D), jnp.float32),
            pltpu.SemaphoreType.DMA,
        ],
    )


def _ffn_body(be_ref, xs_ref, wp_ref, wg_ref, wu_ref, wd_ref,
              ga_ref, gb_ref, ua_ref, ub_ref, da_ref, db_ref, out_ref):
    del be_ref
    xb = xs_ref[...]
    c11 = (((1,), (1,)), ((), ()))
    g = jnp.dot(xb, wg_ref[0], preferred_element_type=jnp.float32)
    xag = lax.dot_general(xb, ga_ref[0], c11,
                          preferred_element_type=jnp.float32)
    g = g + lax.dot_general(xag, gb_ref[0], c11,
                            preferred_element_type=jnp.float32)
    u = jnp.dot(xb, wu_ref[0], preferred_element_type=jnp.float32)
    xau = lax.dot_general(xb, ua_ref[0], c11,
                          preferred_element_type=jnp.float32)
    u = u + lax.dot_general(xau, ub_ref[0], c11,
                            preferred_element_type=jnp.float32)
    h = (g * jax.nn.sigmoid(g)) * u
    y = jnp.dot(h, wd_ref[0], preferred_element_type=jnp.float32)
    had = lax.dot_general(h, da_ref[0], c11,
                          preferred_element_type=jnp.float32)
    y = y + lax.dot_general(had, db_ref[0], c11,
                            preferred_element_type=jnp.float32)
    out_ref[...] = y * wp_ref[...]


def _run_ffn(block_expert, xs, wperm2d, w_gate, w_up, w_down,
             ga, gb, ua, ub, da, db, interpret=False):
    def eix(b, be):
        return (be[b], 0, 0)

    grid_spec = pltpu.PrefetchScalarGridSpec(
        num_scalar_prefetch=1,
        grid=(_NB,),
        in_specs=[
            pl.BlockSpec((_BLK, _D), lambda b, be: (b, 0)),
            pl.BlockSpec((_BLK, 1), lambda b, be: (b, 0)),
            pl.BlockSpec((1, _D, _F), eix),
            pl.BlockSpec((1, _D, _F), eix),
            pl.BlockSpec((1, _F, _D), eix),
            pl.BlockSpec((1, _R, _D), eix),
            pl.BlockSpec((1, _F, _R), eix),
            pl.BlockSpec((1, _R, _D), eix),
            pl.BlockSpec((1, _F, _R), eix),
            pl.BlockSpec((1, _R, _F), eix),
            pl.BlockSpec((1, _D, _R), eix),
        ],
        out_specs=pl.BlockSpec((_BLK, _D), lambda b, be: (b, 0)),
    )
    return pl.pallas_call(
        _ffn_body,
        grid_spec=grid_spec,
        out_shape=jax.ShapeDtypeStruct((_P, _D), jnp.float32),
        compiler_params=pltpu.CompilerParams(
            dimension_semantics=("arbitrary",),
        ),
        interpret=interpret,
    )(block_expert, xs, wperm2d, w_gate, w_up, w_down,
      ga, gb, ua, ub, da, db)


def _sc_combine_body(ys_hbm, pos0_hbm, pos1_hbm, out_hbm,
                     p0_v, p1_v, acc_v, b1_v, sem0, sem1):
    wid = lax.axis_index("s") * _NC + lax.axis_index("c")
    tpw = _T // _NW
    base = wid * tpw
    pltpu.sync_copy(pos0_hbm.at[pl.ds(base, tpw)], p0_v)
    pltpu.sync_copy(pos1_hbm.at[pl.ds(base, tpw)], p1_v)
    cp0 = pltpu.async_copy(ys_hbm.at[p0_v], acc_v, sem0)
    cp1 = pltpu.async_copy(ys_hbm.at[p1_v], b1_v, sem1)
    cp0.wait()
    cp1.wait()

    nv = _D // 16

    @plsc.parallel_loop(0, tpw, unroll=2)
    def _add_tok(t):
        for j in range(nv):
            sl = pl.ds(j * 16, 16)
            acc_v[t, sl] += b1_v[t, sl]

    pltpu.sync_copy(acc_v, out_hbm.at[pl.ds(base, tpw)])


@functools.cache
def _sc_combine():
    return pl.kernel(
        _sc_combine_body,
        out_type=jax.ShapeDtypeStruct((_T, _D), jnp.float32),
        mesh=plsc.VectorSubcoreMesh(core_axis_name="c", subcore_axis_name="s"),
        scratch_types=[
            pltpu.VMEM((_T // _NW,), jnp.int32),
            pltpu.VMEM((_T // _NW,), jnp.int32),
            pltpu.VMEM((_T // _NW, _D), jnp.float32),
            pltpu.VMEM((_T // _NW, _D), jnp.float32),
            pltpu.SemaphoreType.DMA,
            pltpu.SemaphoreType.DMA,
        ],
    )


def kernel(hidden_states, W_router, W_gate, W_up, W_down,
           gate_lora_a, gate_lora_b, up_lora_a, up_lora_b,
           down_lora_a, down_lora_b):
    x = hidden_states.reshape(_T, _D)
    wr_pad = jnp.zeros((_D, _LANES), jnp.float32).at[:, :_E].set(W_router)
    ids_full, ws_full = _run_router(x, wr_pad)
    ids = ids_full[:, :_K]
    ws = ws_full[:, :_K]
    perm, wperm, pos0, pos1, block_expert = _routing_metadata(ids, ws)
    xs = _sc_gather()(x, perm)
    ys = _run_ffn(block_expert, xs, wperm.reshape(_P, 1),
                  W_gate, W_up, W_down,
                  gate_lora_a, gate_lora_b, up_lora_a, up_lora_b,
                  down_lora_a, down_lora_b)
    out = _sc_combine()(ys, pos0, pos1)
    return out.reshape(_B, _S, _D)
